# baseline (speedup 1.0000x reference)
import jax
import jax.numpy as jnp
from jax import lax
from jax.experimental import pallas as pl
from jax.experimental.pallas import tpu as pltpu
from jax.experimental.pallas import tpu_sc as plsc

B = 16384
D = 4096
NC = 2
NS = 16
NW = NC * NS
RPW = B // NW
R = 4
NCH = RPW // R
L = 16


def _body(x_ref, perm_ref, y_ref, ld_ref, perm_v, in_v, out_v, z_v):
    c = lax.axis_index("c")
    s = lax.axis_index("s")
    wid = s * NC + c
    base = wid * RPW

    pltpu.sync_copy(perm_ref, perm_v)

    def zbody(i, carry):
        z_v[pl.ds(i * L, L)] = jnp.zeros((L,), jnp.float32)
        return carry

    lax.fori_loop(0, RPW // L, zbody, 0)
    pltpu.sync_copy(z_v, ld_ref.at[pl.ds(base, RPW)])

    def chunk(g, carry):
        row0 = base + g * R
        pltpu.sync_copy(x_ref.at[pl.ds(row0 * D, R * D)], in_v)

        def jbody(j, carry2):
            idx = perm_v[pl.ds(j * L, L)]
            for r in range(R):
                out_v[pl.ds(r * D + j * L, L)] = plsc.load_gather(
                    in_v, [idx + r * D]
                )
            return carry2

        lax.fori_loop(0, D // L, jbody, 0)
        pltpu.sync_copy(out_v, y_ref.at[pl.ds(row0 * D, R * D)])
        return carry

    lax.fori_loop(0, NCH, chunk, 0)


@jax.jit
def kernel(x, perm):
    mesh = plsc.VectorSubcoreMesh(
        core_axis_name="c", subcore_axis_name="s", num_cores=NC, num_subcores=NS
    )
    f = pl.kernel(
        _body,
        out_type=(
            jax.ShapeDtypeStruct((B * D,), jnp.float32),
            jax.ShapeDtypeStruct((B,), jnp.float32),
        ),
        mesh=mesh,
        compiler_params=pltpu.CompilerParams(needs_layout_passes=False),
        scratch_types=[
            pltpu.VMEM((D,), jnp.int32),
            pltpu.VMEM((R * D,), jnp.float32),
            pltpu.VMEM((R * D,), jnp.float32),
            pltpu.VMEM((RPW,), jnp.float32),
        ],
    )
    y, ld = f(x.reshape(-1), perm)
    return y.reshape(B, D), ld


# speedup vs baseline: 2.1689x; 2.1689x over previous
import jax
import jax.numpy as jnp
from jax import lax
from jax.experimental import pallas as pl
from jax.experimental.pallas import tpu as pltpu
from jax.experimental.pallas import tpu_sc as plsc

B = 16384
D = 4096
NC = 2
NS = 16
NW = NC * NS
RPW = B // NW
R = 4
NCH = RPW // R
L = 16


def _body(x_ref, perm_ref, y_ref, ld_ref,
          perm_v, in0, in1, out0, out1, z_v,
          isem0, isem1, osem0, osem1):
    c = lax.axis_index("c")
    s = lax.axis_index("s")
    wid = s * NC + c
    base = wid * RPW

    pltpu.sync_copy(perm_ref, perm_v)

    @plsc.parallel_loop(0, RPW // L, 1, unroll=8)
    def _zero(i):
        z_v[pl.ds(i * L, L)] = jnp.zeros((L,), jnp.float32)

    pltpu.sync_copy(z_v, ld_ref.at[pl.ds(base, RPW)])

    def in_copy(g, buf, sem):
        off = (base + g * R) * D
        return pltpu.make_async_copy(x_ref.at[pl.ds(off, R * D)], buf, sem)

    def out_copy(g, buf, sem):
        off = (base + g * R) * D
        return pltpu.make_async_copy(buf, y_ref.at[pl.ds(off, R * D)], sem)

    def compute(ibuf, obuf):
        @plsc.parallel_loop(0, D // L, 1, unroll=8)
        def _jbody(j):
            idx = perm_v[pl.ds(j * L, L)]
            for r in range(R):
                obuf[pl.ds(r * D + j * L, L)] = plsc.load_gather(
                    ibuf, [idx + r * D]
                )

    bufs = ((in0, out0, isem0, osem0), (in1, out1, isem1, osem1))

    in_copy(0, in0, isem0).start()
    in_copy(1, in1, isem1).start()

    def gg_body(gg, carry):
        for b, (ibuf, obuf, isem, osem) in enumerate(bufs):
            g = 2 * gg + b
            in_copy(g, ibuf, isem).wait()

            @pl.when(gg > 0)
            def _wait_prev_out():
                out_copy(g - 2, obuf, osem).wait()

            compute(ibuf, obuf)
            out_copy(g, obuf, osem).start()

            @pl.when(g + 2 < NCH)
            def _start_next_in():
                in_copy(g + 2, ibuf, isem).start()

        return carry

    lax.fori_loop(0, NCH // 2, gg_body, 0)

    out_copy(NCH - 2, out0, osem0).wait()
    out_copy(NCH - 1, out1, osem1).wait()


@jax.jit
def kernel(x, perm):
    mesh = plsc.VectorSubcoreMesh(
        core_axis_name="c", subcore_axis_name="s", num_cores=NC, num_subcores=NS
    )
    f = pl.kernel(
        _body,
        out_type=(
            jax.ShapeDtypeStruct((B * D,), jnp.float32),
            jax.ShapeDtypeStruct((B,), jnp.float32),
        ),
        mesh=mesh,
        compiler_params=pltpu.CompilerParams(needs_layout_passes=False),
        scratch_types=[
            pltpu.VMEM((D,), jnp.int32),
            pltpu.VMEM((R * D,), jnp.float32),
            pltpu.VMEM((R * D,), jnp.float32),
            pltpu.VMEM((R * D,), jnp.float32),
            pltpu.VMEM((R * D,), jnp.float32),
            pltpu.VMEM((RPW,), jnp.float32),
            pltpu.SemaphoreType.DMA,
            pltpu.SemaphoreType.DMA,
            pltpu.SemaphoreType.DMA,
            pltpu.SemaphoreType.DMA,
        ],
    )
    y, ld = f(x.reshape(-1), perm)
    return y.reshape(B, D), ld


# speedup vs baseline: 6.9502x; 3.2044x over previous
import jax
import jax.numpy as jnp
from jax import lax
from jax.experimental import pallas as pl
from jax.experimental.pallas import tpu as pltpu
from jax.experimental.pallas import tpu_sc as plsc

B = 16384
D = 4096
NC = 2
NS = 16
NW = NC * NS
RPW = B // NW
R = 8
NCH = RPW // R
L = 16
DH = D // 2


def _body(x_ref, perm_ref, y_ref, ld_ref,
          perm_v, in0, in1, out0, out1, z_v,
          isem0, isem1, osem0, osem1):
    c = lax.axis_index("c")
    s = lax.axis_index("s")
    wid = s * NC + c
    base = wid * RPW

    pltpu.sync_copy(perm_ref, perm_v)

    @plsc.parallel_loop(0, RPW // L, 1, unroll=8)
    def _zero(i):
        z_v[pl.ds(i * L, L)] = jnp.zeros((L,), jnp.float32)

    pltpu.sync_copy(z_v, ld_ref.at[pl.ds(base, RPW)])

    def in_copy(g, buf, sem):
        row0 = base + g * R
        return pltpu.make_async_copy(x_ref.at[pl.ds(row0, R), :], buf, sem)

    def out_copy(g, h, buf, sem):
        row0 = base + g * R
        return pltpu.make_async_copy(
            buf, y_ref.at[pl.ds(row0, R), pl.ds(h * DH, DH)], sem)

    def compute(ibuf, obuf, h):
        @plsc.parallel_loop(0, DH // L, 1, unroll=8)
        def _jbody(jj):
            idx = perm_v[pl.ds(h * DH + jj * L, L)]
            for r in range(R):
                rvec = jnp.full((L,), r, jnp.int32)
                obuf[r, pl.ds(jj * L, L)] = plsc.load_gather(
                    ibuf, [rvec, idx]
                )

    obufs = ((out0, osem0), (out1, osem1))
    ibufs = ((in0, isem0), (in1, isem1))

    in_copy(0, in0, isem0).start()
    in_copy(1, in1, isem1).start()

    def gg_body(gg, carry):
        for b, (ibuf, isem) in enumerate(ibufs):
            g = 2 * gg + b
            in_copy(g, ibuf, isem).wait()

            for h, (obuf, osem) in enumerate(obufs):
                if b == 0:
                    @pl.when(gg > 0)
                    def _wait_prev_out():
                        out_copy(g - 1, h, obuf, osem).wait()
                else:
                    out_copy(g - 1, h, obuf, osem).wait()
                compute(ibuf, obuf, h)
                out_copy(g, h, obuf, osem).start()

            @pl.when(g + 2 < NCH)
            def _start_next_in():
                in_copy(g + 2, ibuf, isem).start()

        return carry

    lax.fori_loop(0, NCH // 2, gg_body, 0)

    out_copy(NCH - 1, 0, out0, osem0).wait()
    out_copy(NCH - 1, 1, out1, osem1).wait()


@jax.jit
def kernel(x, perm):
    mesh = plsc.VectorSubcoreMesh(
        core_axis_name="c", subcore_axis_name="s", num_cores=NC, num_subcores=NS
    )
    f = pl.kernel(
        _body,
        out_type=(
            jax.ShapeDtypeStruct((B, D), jnp.float32),
            jax.ShapeDtypeStruct((B,), jnp.float32),
        ),
        mesh=mesh,
        compiler_params=pltpu.CompilerParams(
            needs_layout_passes=False, use_tc_tiling_on_sc=True
        ),
        scratch_types=[
            pltpu.VMEM((D,), jnp.int32),
            pltpu.VMEM((R, D), jnp.float32),
            pltpu.VMEM((R, D), jnp.float32),
            pltpu.VMEM((R, DH), jnp.float32),
            pltpu.VMEM((R, DH), jnp.float32),
            pltpu.VMEM((RPW,), jnp.float32),
            pltpu.SemaphoreType.DMA,
            pltpu.SemaphoreType.DMA,
            pltpu.SemaphoreType.DMA,
            pltpu.SemaphoreType.DMA,
        ],
    )
    return f(x, perm)
